# Initial kernel scaffold; baseline (speedup 1.0000x reference)
import jax
import jax.numpy as jnp
from jax.experimental import pallas as pl


def kernel(z_e, emb_w):
    raise NotImplementedError("write your pallas kernel here")


# speedup vs baseline: 1.1368x; 1.1368x over previous
import jax
import jax.numpy as jnp
from jax.experimental import pallas as pl

_K = 512
_D = 64
_B, _C, _H, _W = 64, 64, 32, 32
_HW = _H * _W
_N = _B * _HW


def _stats_kernel(z_ref, sum_ref, sq_ref):
    x = z_ref[0].reshape(_C, _HW)
    sum_ref[0] = jnp.sum(x, axis=1).reshape(1, _C)
    sq_ref[0] = jnp.sum(x * x, axis=1).reshape(1, _C)


def _main_kernel(z_ref, std_ref, emb_ref, zq_ref, idx_ref, cnt_ref, loss_ref):
    x = z_ref[0].reshape(_C, _HW)
    std = std_ref[0, 0]
    xn = x / std[:, None]
    zt = xn.T
    e = emb_ref[...]
    scores = jnp.dot(zt.astype(jnp.bfloat16), e.T.astype(jnp.bfloat16),
                     preferred_element_type=jnp.float32)
    row_sq = jnp.sum(zt * zt, axis=1, keepdims=True)
    e_sq = jnp.sum(e * e, axis=1)
    dist = row_sq - 2.0 * scores + e_sq[None, :]
    min_d = jnp.min(dist, axis=1, keepdims=True)
    iota_k = jax.lax.broadcasted_iota(jnp.int32, (_HW, _K), 1)
    idx = jnp.min(jnp.where(dist == min_d, iota_k, _K), axis=1).astype(jnp.int32)
    onehot = (iota_k == idx[:, None]).astype(jnp.float32)
    zq_t = jnp.dot(onehot, e, preferred_element_type=jnp.float32)
    zq = zq_t.T
    zq_st = xn + (zq - xn)
    zq_ref[0] = zq_st.reshape(_C, _H, _W)
    idx_ref[0] = idx.reshape(1, _HW)
    cnt_ref[0] = jnp.sum(onehot, axis=0).reshape(1, _K)
    loss_ref[0] = jnp.full((1, 128), jnp.sum((xn - zq) ** 2), dtype=jnp.float32)


def kernel(z_e, emb_w):
    eps = 1e-5
    z32 = z_e.astype(jnp.float32)
    emb = emb_w.astype(jnp.float32)

    sums, sqs = pl.pallas_call(
        _stats_kernel,
        grid=(_B,),
        in_specs=[pl.BlockSpec((1, _C, _H, _W), lambda b: (b, 0, 0, 0))],
        out_specs=[
            pl.BlockSpec((1, 1, _C), lambda b: (b, 0, 0)),
            pl.BlockSpec((1, 1, _C), lambda b: (b, 0, 0)),
        ],
        out_shape=[
            jax.ShapeDtypeStruct((_B, 1, _C), jnp.float32),
            jax.ShapeDtypeStruct((_B, 1, _C), jnp.float32),
        ],
    )(z32)

    total = jnp.sum(sums, axis=(0, 1))
    sq_total = jnp.sum(sqs, axis=(0, 1))
    mean = total / _N
    var = (sq_total - _N * mean * mean) / (_N - 1)
    std = jnp.clip(jnp.sqrt(var), eps, None)
    std2 = std.reshape(1, 1, _C)

    zq, idx, cnts, loss_parts = pl.pallas_call(
        _main_kernel,
        grid=(_B,),
        in_specs=[
            pl.BlockSpec((1, _C, _H, _W), lambda b: (b, 0, 0, 0)),
            pl.BlockSpec((1, 1, _C), lambda b: (0, 0, 0)),
            pl.BlockSpec((_K, _D), lambda b: (0, 0)),
        ],
        out_specs=[
            pl.BlockSpec((1, _C, _H, _W), lambda b: (b, 0, 0, 0)),
            pl.BlockSpec((1, 1, _HW), lambda b: (b, 0, 0)),
            pl.BlockSpec((1, 1, _K), lambda b: (b, 0, 0)),
            pl.BlockSpec((1, 1, 128), lambda b: (b, 0, 0)),
        ],
        out_shape=[
            jax.ShapeDtypeStruct((_B, _C, _H, _W), jnp.float32),
            jax.ShapeDtypeStruct((_B, 1, _HW), jnp.int32),
            jax.ShapeDtypeStruct((_B, 1, _K), jnp.float32),
            jax.ShapeDtypeStruct((_B, 1, 128), jnp.float32),
        ],
    )(z32, std2, emb)

    counts = jnp.sum(cnts, axis=(0, 1))
    avg_probs = counts / _N
    perplexity = jnp.exp(-jnp.sum(avg_probs * jnp.log(jnp.clip(avg_probs, 1e-10, None))))

    mse = jnp.sum(loss_parts[:, 0, 0]) / (_N * _D)
    vq_loss = 0.25 * mse + mse

    z_q_st = zq.astype(z_e.dtype)
    indices = idx.reshape(_B, _H, _W)
    return (z_q_st, vq_loss, perplexity, indices)


# speedup vs baseline: 1.3525x; 1.1898x over previous
import jax
import jax.numpy as jnp
from jax.experimental import pallas as pl

_K = 512
_D = 64
_B, _C, _H, _W = 64, 64, 32, 32
_HW = _H * _W
_N = _B * _HW


def _stats_kernel(z_ref, sum_ref, sq_ref):
    x = z_ref[0].reshape(_C, _HW)
    sum_ref[0] = jnp.sum(x, axis=1).reshape(1, _C)
    sq_ref[0] = jnp.sum(x * x, axis=1).reshape(1, _C)


def _main_kernel(z_ref, std_ref, emb_ref, zq_ref, idx_ref, cnt_ref, loss_ref):
    x = z_ref[0].reshape(_C, _HW)
    std = std_ref[0, 0]
    xn = x / std[:, None]
    e = emb_ref[...]
    scores = jnp.dot(e.astype(jnp.bfloat16), xn.astype(jnp.bfloat16),
                     preferred_element_type=jnp.float32)
    row_sq = jnp.sum(xn * xn, axis=0, keepdims=True)
    e_sq = jnp.sum(e * e, axis=1, keepdims=True)
    dist = row_sq - 2.0 * scores + e_sq
    min_d = jnp.min(dist, axis=0, keepdims=True)
    iota_k = jax.lax.broadcasted_iota(jnp.int32, (_K, _HW), 0)
    idx = jnp.min(jnp.where(dist == min_d, iota_k, _K), axis=0).astype(jnp.int32)
    onehot = (iota_k == idx[None, :]).astype(jnp.float32)
    zq = jax.lax.dot_general(e, onehot, (((0,), (0,)), ((), ())),
                             preferred_element_type=jnp.float32)
    zq_st = xn + (zq - xn)
    zq_ref[0] = zq_st.reshape(_C, _H, _W)
    idx_ref[0] = idx.reshape(1, _HW)
    cnt_ref[0] = jnp.sum(onehot, axis=1).reshape(1, _K)
    loss_ref[0] = min_d


def kernel(z_e, emb_w):
    eps = 1e-5
    z32 = z_e.astype(jnp.float32)
    emb = emb_w.astype(jnp.float32)

    sums, sqs = pl.pallas_call(
        _stats_kernel,
        grid=(_B,),
        in_specs=[pl.BlockSpec((1, _C, _H, _W), lambda b: (b, 0, 0, 0))],
        out_specs=[
            pl.BlockSpec((1, 1, _C), lambda b: (b, 0, 0)),
            pl.BlockSpec((1, 1, _C), lambda b: (b, 0, 0)),
        ],
        out_shape=[
            jax.ShapeDtypeStruct((_B, 1, _C), jnp.float32),
            jax.ShapeDtypeStruct((_B, 1, _C), jnp.float32),
        ],
    )(z32)

    total = jnp.sum(sums, axis=(0, 1))
    sq_total = jnp.sum(sqs, axis=(0, 1))
    mean = total / _N
    var = (sq_total - _N * mean * mean) / (_N - 1)
    std = jnp.clip(jnp.sqrt(var), eps, None)
    std2 = std.reshape(1, 1, _C)

    zq, idx, cnts, loss_parts = pl.pallas_call(
        _main_kernel,
        grid=(_B,),
        in_specs=[
            pl.BlockSpec((1, _C, _H, _W), lambda b: (b, 0, 0, 0)),
            pl.BlockSpec((1, 1, _C), lambda b: (0, 0, 0)),
            pl.BlockSpec((_K, _D), lambda b: (0, 0)),
        ],
        out_specs=[
            pl.BlockSpec((1, _C, _H, _W), lambda b: (b, 0, 0, 0)),
            pl.BlockSpec((1, 1, _HW), lambda b: (b, 0, 0)),
            pl.BlockSpec((1, 1, _K), lambda b: (b, 0, 0)),
            pl.BlockSpec((1, 1, _HW), lambda b: (b, 0, 0)),
        ],
        out_shape=[
            jax.ShapeDtypeStruct((_B, _C, _H, _W), jnp.float32),
            jax.ShapeDtypeStruct((_B, 1, _HW), jnp.int32),
            jax.ShapeDtypeStruct((_B, 1, _K), jnp.float32),
            jax.ShapeDtypeStruct((_B, 1, _HW), jnp.float32),
        ],
    )(z32, std2, emb)

    counts = jnp.sum(cnts, axis=(0, 1))
    avg_probs = counts / _N
    perplexity = jnp.exp(-jnp.sum(avg_probs * jnp.log(jnp.clip(avg_probs, 1e-10, None))))

    mse = jnp.sum(loss_parts) / (_N * _D)
    vq_loss = 0.25 * mse + mse

    z_q_st = zq.astype(z_e.dtype)
    indices = idx.reshape(_B, _H, _W)
    return (z_q_st, vq_loss, perplexity, indices)


# speedup vs baseline: 1.5351x; 1.1350x over previous
import jax
import jax.numpy as jnp
from jax.experimental import pallas as pl

_K = 512
_D = 64
_B, _C, _H, _W = 64, 64, 32, 32
_HW = _H * _W
_N = _B * _HW


_SPB = 4


def _stats_kernel(z_ref, sum_ref, sq_ref):
    for i in range(_SPB):
        x = z_ref[i].reshape(_C, _HW)
        sum_ref[i] = jnp.sum(x, axis=1).reshape(1, _C)
        sq_ref[i] = jnp.sum(x * x, axis=1).reshape(1, _C)


_BPB = 1
_L = _BPB * _HW


def _main_kernel(z_ref, std_ref, emb_ref, zq_ref, idx_ref, cnt_ref, loss_ref):
    x = jnp.concatenate([z_ref[i].reshape(_C, _HW) for i in range(_BPB)],
                        axis=1)
    std = std_ref[0, 0]
    xn = x / std[:, None]
    e = emb_ref[...]
    e_bf = e.astype(jnp.bfloat16)
    xn_bf = xn.astype(jnp.bfloat16)
    scores_m2 = jnp.dot((e * -2.0).astype(jnp.bfloat16), xn_bf,
                        preferred_element_type=jnp.float32)
    t = xn * xn
    while t.shape[0] > 1:
        h = t.shape[0] // 2
        t = t[:h] + t[h:]
    row_sq = t
    e_sq = jnp.sum(e * e, axis=1, keepdims=True)
    dist = (row_sq + scores_m2) + e_sq
    min_d = jnp.min(dist, axis=0, keepdims=True)
    iota_k = jax.lax.broadcasted_iota(jnp.int32, (_K, _L), 0)
    idx = jnp.min(jnp.where(dist == min_d, iota_k, _K), axis=0).astype(jnp.int32)
    onehot = (iota_k == idx[None, :]).astype(jnp.bfloat16)
    zq = jax.lax.dot_general(e_bf, onehot, (((0,), (0,)), ((), ())),
                             preferred_element_type=jnp.float32)
    cnt_ref[0] = jax.lax.dot_general(
        jnp.ones((1, _L), jnp.bfloat16), onehot, (((1,), (1,)), ((), ())),
        preferred_element_type=jnp.float32)
    zq_st = xn + (zq - xn)
    for i in range(_BPB):
        zq_ref[i] = zq_st[:, i * _HW:(i + 1) * _HW].reshape(_C, _H, _W)
    idx_ref[0] = idx.reshape(1, _L)
    loss_ref[0] = min_d


def kernel(z_e, emb_w):
    eps = 1e-5
    z32 = z_e.astype(jnp.float32)
    emb = emb_w.astype(jnp.float32)

    sums, sqs = pl.pallas_call(
        _stats_kernel,
        grid=(_B // _SPB,),
        in_specs=[pl.BlockSpec((_SPB, _C, _H, _W), lambda b: (b, 0, 0, 0))],
        out_specs=[
            pl.BlockSpec((_SPB, 1, _C), lambda b: (b, 0, 0)),
            pl.BlockSpec((_SPB, 1, _C), lambda b: (b, 0, 0)),
        ],
        out_shape=[
            jax.ShapeDtypeStruct((_B, 1, _C), jnp.float32),
            jax.ShapeDtypeStruct((_B, 1, _C), jnp.float32),
        ],
    )(z32)

    total = jnp.sum(sums, axis=(0, 1))
    sq_total = jnp.sum(sqs, axis=(0, 1))
    mean = total / _N
    var = (sq_total - _N * mean * mean) / (_N - 1)
    std = jnp.clip(jnp.sqrt(var), eps, None)
    std2 = std.reshape(1, 1, _C)

    n_steps = _B // _BPB
    zq, idx, cnts, loss_parts = pl.pallas_call(
        _main_kernel,
        grid=(n_steps,),
        in_specs=[
            pl.BlockSpec((_BPB, _C, _H, _W), lambda b: (b, 0, 0, 0)),
            pl.BlockSpec((1, 1, _C), lambda b: (0, 0, 0)),
            pl.BlockSpec((_K, _D), lambda b: (0, 0)),
        ],
        out_specs=[
            pl.BlockSpec((_BPB, _C, _H, _W), lambda b: (b, 0, 0, 0)),
            pl.BlockSpec((1, 1, _L), lambda b: (b, 0, 0)),
            pl.BlockSpec((1, 1, _K), lambda b: (b, 0, 0)),
            pl.BlockSpec((1, 1, _L), lambda b: (b, 0, 0)),
        ],
        out_shape=[
            jax.ShapeDtypeStruct((_B, _C, _H, _W), jnp.float32),
            jax.ShapeDtypeStruct((n_steps, 1, _L), jnp.int32),
            jax.ShapeDtypeStruct((n_steps, 1, _K), jnp.float32),
            jax.ShapeDtypeStruct((n_steps, 1, _L), jnp.float32),
        ],
    )(z32, std2, emb)

    counts = jnp.sum(cnts, axis=(0, 1))
    avg_probs = counts / _N
    perplexity = jnp.exp(-jnp.sum(avg_probs * jnp.log(jnp.clip(avg_probs, 1e-10, None))))

    mse = jnp.sum(loss_parts) / (_N * _D)
    vq_loss = 0.25 * mse + mse

    z_q_st = zq.astype(z_e.dtype)
    indices = idx.reshape(_B, _H, _W)
    return (z_q_st, vq_loss, perplexity, indices)


# speedup vs baseline: 1.5469x; 1.0077x over previous
import jax
import jax.numpy as jnp
from jax.experimental import pallas as pl

_K = 512
_D = 64
_B, _C, _H, _W = 64, 64, 32, 32
_HW = _H * _W
_N = _B * _HW


_SPB = 4


def _stats_kernel(z_ref, sum_ref, sq_ref):
    for i in range(_SPB):
        x = z_ref[i].reshape(_C, _HW)
        sum_ref[i] = jnp.sum(x, axis=1).reshape(1, _C)
        sq_ref[i] = jnp.sum(x * x, axis=1).reshape(1, _C)


_BPB = 2


def _main_kernel(z_ref, std_ref, emb_ref, zq_ref, idx_ref, cnt_ref, loss_ref):
    std = std_ref[0, 0]
    e = emb_ref[...]
    e_bf = e.astype(jnp.bfloat16)
    em2_bf = (e * -2.0).astype(jnp.bfloat16)
    e_sq = jnp.sum(e * e, axis=1, keepdims=True)
    iota_k = jax.lax.broadcasted_iota(jnp.int32, (_K, _HW), 0)
    ones_row = jnp.ones((1, _HW), jnp.bfloat16)
    cnt = jnp.zeros((1, _K), jnp.float32)
    for i in range(_BPB):
        x = z_ref[i].reshape(_C, _HW)
        xn = x / std[:, None]
        scores_m2 = jnp.dot(em2_bf, xn.astype(jnp.bfloat16),
                            preferred_element_type=jnp.float32)
        t = xn * xn
        while t.shape[0] > 1:
            h = t.shape[0] // 2
            t = t[:h] + t[h:]
        row_sq = t
        dist = (row_sq + scores_m2) + e_sq
        min_d = jnp.min(dist, axis=0, keepdims=True)
        idx = jnp.min(jnp.where(dist == min_d, iota_k, _K),
                      axis=0).astype(jnp.int32)
        onehot = (iota_k == idx[None, :]).astype(jnp.bfloat16)
        zq = jax.lax.dot_general(e_bf, onehot, (((0,), (0,)), ((), ())),
                                 preferred_element_type=jnp.float32)
        cnt = cnt + jax.lax.dot_general(
            ones_row, onehot, (((1,), (1,)), ((), ())),
            preferred_element_type=jnp.float32)
        zq_ref[i] = (xn + (zq - xn)).reshape(_C, _H, _W)
        idx_ref[0, i] = idx
        loss_ref[0, i] = min_d[0]
    cnt_ref[0] = cnt


def kernel(z_e, emb_w):
    eps = 1e-5
    z32 = z_e.astype(jnp.float32)
    emb = emb_w.astype(jnp.float32)

    sums, sqs = pl.pallas_call(
        _stats_kernel,
        grid=(_B // _SPB,),
        in_specs=[pl.BlockSpec((_SPB, _C, _H, _W), lambda b: (b, 0, 0, 0))],
        out_specs=[
            pl.BlockSpec((_SPB, 1, _C), lambda b: (b, 0, 0)),
            pl.BlockSpec((_SPB, 1, _C), lambda b: (b, 0, 0)),
        ],
        out_shape=[
            jax.ShapeDtypeStruct((_B, 1, _C), jnp.float32),
            jax.ShapeDtypeStruct((_B, 1, _C), jnp.float32),
        ],
    )(z32)

    total = jnp.sum(sums, axis=(0, 1))
    sq_total = jnp.sum(sqs, axis=(0, 1))
    mean = total / _N
    var = (sq_total - _N * mean * mean) / (_N - 1)
    std = jnp.clip(jnp.sqrt(var), eps, None)
    std2 = std.reshape(1, 1, _C)

    n_steps = _B // _BPB
    zq, idx, cnts, loss_parts = pl.pallas_call(
        _main_kernel,
        grid=(n_steps,),
        in_specs=[
            pl.BlockSpec((_BPB, _C, _H, _W), lambda b: (b, 0, 0, 0)),
            pl.BlockSpec((1, 1, _C), lambda b: (0, 0, 0)),
            pl.BlockSpec((_K, _D), lambda b: (0, 0)),
        ],
        out_specs=[
            pl.BlockSpec((_BPB, _C, _H, _W), lambda b: (b, 0, 0, 0)),
            pl.BlockSpec((1, _BPB, _HW), lambda b: (b, 0, 0)),
            pl.BlockSpec((1, 1, _K), lambda b: (b, 0, 0)),
            pl.BlockSpec((1, _BPB, _HW), lambda b: (b, 0, 0)),
        ],
        out_shape=[
            jax.ShapeDtypeStruct((_B, _C, _H, _W), jnp.float32),
            jax.ShapeDtypeStruct((n_steps, _BPB, _HW), jnp.int32),
            jax.ShapeDtypeStruct((n_steps, 1, _K), jnp.float32),
            jax.ShapeDtypeStruct((n_steps, _BPB, _HW), jnp.float32),
        ],
    )(z32, std2, emb)

    counts = jnp.sum(cnts, axis=(0, 1))
    avg_probs = counts / _N
    perplexity = jnp.exp(-jnp.sum(avg_probs * jnp.log(jnp.clip(avg_probs, 1e-10, None))))

    mse = jnp.sum(loss_parts) / (_N * _D)
    vq_loss = 0.25 * mse + mse

    z_q_st = zq.astype(z_e.dtype)
    indices = idx.reshape(_B, _H, _W)
    return (z_q_st, vq_loss, perplexity, indices)


# speedup vs baseline: 2.4506x; 1.5842x over previous
import jax
import jax.numpy as jnp
from jax.experimental import pallas as pl

_K = 512
_D = 64
_B, _C, _H, _W = 64, 64, 32, 32
_HW = _H * _W
_N = _B * _HW


_SPB = 4


def _stats_kernel(z_ref, sum_ref, sq_ref):
    for i in range(_SPB):
        x = z_ref[i]
        sum_ref[i] = jnp.sum(x, axis=1).reshape(1, _C)
        sq_ref[i] = jnp.sum(x * x, axis=1).reshape(1, _C)


_BPB = 2


def _main_kernel(z_ref, std_ref, emb_ref, zq_ref, idx_ref, cnt_ref, loss_ref):
    std = std_ref[0, 0]
    e = emb_ref[...]
    e_bf = e.astype(jnp.bfloat16)
    em2_bf = (e * -2.0).astype(jnp.bfloat16)
    e_sq = jnp.sum(e * e, axis=1, keepdims=True)
    iota_k = jax.lax.broadcasted_iota(jnp.int32, (_K, _HW), 0)
    ones_row = jnp.ones((1, _HW), jnp.bfloat16)
    cnt = jnp.zeros((1, _K), jnp.float32)
    for i in range(_BPB):
        x = z_ref[i]
        xn = x / std[:, None]
        scores_m2 = jnp.dot(em2_bf, xn.astype(jnp.bfloat16),
                            preferred_element_type=jnp.float32)
        t = xn * xn
        while t.shape[0] > 1:
            h = t.shape[0] // 2
            t = t[:h] + t[h:]
        row_sq = t
        dist = (row_sq + scores_m2) + e_sq
        min_d = jnp.min(dist, axis=0, keepdims=True)
        idx = jnp.min(jnp.where(dist == min_d, iota_k, _K),
                      axis=0).astype(jnp.int32)
        onehot = (iota_k == idx[None, :]).astype(jnp.bfloat16)
        zq = jax.lax.dot_general(e_bf, onehot, (((0,), (0,)), ((), ())),
                                 preferred_element_type=jnp.float32)
        cnt = cnt + jax.lax.dot_general(
            ones_row, onehot, (((1,), (1,)), ((), ())),
            preferred_element_type=jnp.float32)
        zq_ref[i] = xn + (zq - xn)
        idx_ref[0, i] = idx
        loss_ref[0, i] = min_d[0]
    cnt_ref[0] = cnt


def kernel(z_e, emb_w):
    eps = 1e-5
    z32 = z_e.astype(jnp.float32).reshape(_B, _C, _HW)
    emb = emb_w.astype(jnp.float32)

    sums, sqs = pl.pallas_call(
        _stats_kernel,
        grid=(_B // _SPB,),
        in_specs=[pl.BlockSpec((_SPB, _C, _HW), lambda b: (b, 0, 0))],
        out_specs=[
            pl.BlockSpec((_SPB, 1, _C), lambda b: (b, 0, 0)),
            pl.BlockSpec((_SPB, 1, _C), lambda b: (b, 0, 0)),
        ],
        out_shape=[
            jax.ShapeDtypeStruct((_B, 1, _C), jnp.float32),
            jax.ShapeDtypeStruct((_B, 1, _C), jnp.float32),
        ],
    )(z32)

    total = jnp.sum(sums, axis=(0, 1))
    sq_total = jnp.sum(sqs, axis=(0, 1))
    mean = total / _N
    var = (sq_total - _N * mean * mean) / (_N - 1)
    std = jnp.clip(jnp.sqrt(var), eps, None)
    std2 = std.reshape(1, 1, _C)

    n_steps = _B // _BPB
    zq, idx, cnts, loss_parts = pl.pallas_call(
        _main_kernel,
        grid=(n_steps,),
        in_specs=[
            pl.BlockSpec((_BPB, _C, _HW), lambda b: (b, 0, 0)),
            pl.BlockSpec((1, 1, _C), lambda b: (0, 0, 0)),
            pl.BlockSpec((_K, _D), lambda b: (0, 0)),
        ],
        out_specs=[
            pl.BlockSpec((_BPB, _C, _HW), lambda b: (b, 0, 0)),
            pl.BlockSpec((1, _BPB, _HW), lambda b: (b, 0, 0)),
            pl.BlockSpec((1, 1, _K), lambda b: (b, 0, 0)),
            pl.BlockSpec((1, _BPB, _HW), lambda b: (b, 0, 0)),
        ],
        out_shape=[
            jax.ShapeDtypeStruct((_B, _C, _HW), jnp.float32),
            jax.ShapeDtypeStruct((n_steps, _BPB, _HW), jnp.int32),
            jax.ShapeDtypeStruct((n_steps, 1, _K), jnp.float32),
            jax.ShapeDtypeStruct((n_steps, _BPB, _HW), jnp.float32),
        ],
    )(z32, std2, emb)

    counts = jnp.sum(cnts, axis=(0, 1))
    avg_probs = counts / _N
    perplexity = jnp.exp(-jnp.sum(avg_probs * jnp.log(jnp.clip(avg_probs, 1e-10, None))))

    mse = jnp.sum(loss_parts) / (_N * _D)
    vq_loss = 0.25 * mse + mse

    z_q_st = zq.reshape(_B, _C, _H, _W).astype(z_e.dtype)
    indices = idx.reshape(_B, _H, _W)
    return (z_q_st, vq_loss, perplexity, indices)
